# Initial kernel scaffold; baseline (speedup 1.0000x reference)
import jax
import jax.numpy as jnp
from jax.experimental import pallas as pl


def kernel(x, adj, W1, b1, W2, b2, W3, b3, W4, b4, W5, b5, Wl1, bl1, Wl2, bl2, Wl3, bl3):
    raise NotImplementedError("write your pallas kernel here")


# speedup vs baseline: 1.1050x; 1.1050x over previous
import functools

import jax
import jax.numpy as jnp
from jax.experimental import pallas as pl
from jax.experimental.pallas import tpu as pltpu

N = 10000
D = 64
BR1 = 200
BR = 400
EPS = 1e-5
NN = 70
BF = jnp.bfloat16


def _compute_y(r_ref, w_ref, m_ref, v_ref, y_ref, normalize):
    r = r_ref[...]
    if normalize:
        h = (r - m_ref[...]) / jnp.sqrt(v_ref[...] + EPS)
    else:
        h = r
    y_ref[...] = jnp.dot(h.astype(BF), w_ref[...].astype(BF),
                         preferred_element_type=jnp.float32)


def _layer1_body(r_ref, adj_ref, w_ref, b_ref, out_ref, adjbf_ref, y_ref):
    @pl.when(pl.program_id(0) == 0)
    def _():
        _compute_y(r_ref, w_ref, None, None, y_ref, normalize=False)

    a16 = adj_ref[...].astype(BF)
    adjbf_ref[...] = a16
    out_ref[...] = jnp.maximum(
        jnp.dot(a16, y_ref[...].astype(BF),
                preferred_element_type=jnp.float32) + b_ref[...],
        0.0,
    )


def _layer_body(r_ref, m_ref, v_ref, adj_ref, w_ref, b_ref, out_ref, y_ref):
    @pl.when(pl.program_id(0) == 0)
    def _():
        _compute_y(r_ref, w_ref, m_ref, v_ref, y_ref, normalize=True)

    out_ref[...] = jnp.maximum(
        jnp.dot(adj_ref[...], y_ref[...].astype(BF),
                preferred_element_type=jnp.float32) + b_ref[...],
        0.0,
    )


def _gcn_layer1(h0, adj, W, b):
    return pl.pallas_call(
        _layer1_body,
        grid=(N // BR1,),
        in_specs=[
            pl.BlockSpec((N, D), lambda i: (0, 0)),
            pl.BlockSpec((BR1, N), lambda i: (i, 0)),
            pl.BlockSpec((D, D), lambda i: (0, 0)),
            pl.BlockSpec((1, D), lambda i: (0, 0)),
        ],
        out_specs=[
            pl.BlockSpec((BR1, D), lambda i: (i, 0)),
            pl.BlockSpec((BR1, N), lambda i: (i, 0)),
        ],
        out_shape=[
            jax.ShapeDtypeStruct((N, D), jnp.float32),
            jax.ShapeDtypeStruct((N, N), BF),
        ],
        scratch_shapes=[pltpu.VMEM((N, D), jnp.float32)],
    )(h0, adj, W, b)


def _gcn_layer(r_prev, adj_bf, W, b):
    m = jnp.mean(r_prev, axis=0, keepdims=True)
    v = jnp.var(r_prev, axis=0, keepdims=True)
    return pl.pallas_call(
        _layer_body,
        grid=(N // BR,),
        in_specs=[
            pl.BlockSpec((N, D), lambda i: (0, 0)),
            pl.BlockSpec((1, D), lambda i: (0, 0)),
            pl.BlockSpec((1, D), lambda i: (0, 0)),
            pl.BlockSpec((BR, N), lambda i: (i, 0)),
            pl.BlockSpec((D, D), lambda i: (0, 0)),
            pl.BlockSpec((1, D), lambda i: (0, 0)),
        ],
        out_specs=pl.BlockSpec((BR, D), lambda i: (i, 0)),
        out_shape=jax.ShapeDtypeStruct((N, D), jnp.float32),
        scratch_shapes=[pltpu.VMEM((N, D), jnp.float32)],
    )(r_prev, m, v, adj_bf, W, b)


BM = 2000
NMBLK = N // BM


def _mlp_body(r_ref, x_ref, w1_ref, b1_ref, w2_ref, b2_ref, w3_ref, b3_ref,
              out_ref):
    hcat = jnp.concatenate([r_ref[...], x_ref[...]], axis=1)
    h1 = jnp.maximum(
        jnp.dot(hcat.astype(BF), w1_ref[...].astype(BF),
                preferred_element_type=jnp.float32) + b1_ref[...],
        0.0,
    )
    h2 = jnp.maximum(
        jnp.dot(h1.astype(BF), w2_ref[...].astype(BF),
                preferred_element_type=jnp.float32) + b2_ref[...],
        0.0,
    )
    s = jnp.dot(h2.astype(BF), w3_ref[...].astype(BF),
                preferred_element_type=jnp.float32)
    out_ref[...] = s[:, :1] + b3_ref[...]


def _mlp_head(r5, x_hi, Wl1, bl1, Wl2, bl2, Wl3p, bl3):
    return pl.pallas_call(
        _mlp_body,
        grid=(NMBLK,),
        in_specs=[
            pl.BlockSpec((BM, D), lambda i: (i, 0)),
            pl.BlockSpec((BM, D), lambda i: (i, 0)),
            pl.BlockSpec((2 * D, 256), lambda i: (0, 0)),
            pl.BlockSpec((1, 256), lambda i: (0, 0)),
            pl.BlockSpec((256, 2 * D), lambda i: (0, 0)),
            pl.BlockSpec((1, 2 * D), lambda i: (0, 0)),
            pl.BlockSpec((2 * D, 2 * D), lambda i: (0, 0)),
            pl.BlockSpec((1, 1), lambda i: (0, 0)),
        ],
        out_specs=pl.BlockSpec((BM, 1), lambda i: (i, 0)),
        out_shape=jax.ShapeDtypeStruct((N, 1), jnp.float32),
    )(r5, x_hi, Wl1, bl1, Wl2, bl2, Wl3p, bl3)


ROWS = 8
COLS = N // ROWS


def _topk_body(s_ref, vals_ref, idx_ref):
    s = s_ref[...]
    iota_r = jax.lax.broadcasted_iota(jnp.int32, (ROWS, COLS), 0)
    iota_c = jax.lax.broadcasted_iota(jnp.int32, (ROWS, COLS), 1)
    idx0 = iota_r * COLS + iota_c
    lane = jax.lax.broadcasted_iota(jnp.int32, (1, 128), 1)

    def step(k, carry):
        s_cur, vacc, iacc = carry
        mx = jnp.max(s_cur)
        cand = jnp.where(s_cur == mx, idx0, jnp.int32(2**31 - 1))
        arg = jnp.min(cand)
        vacc = jnp.where(lane == k, mx, vacc)
        iacc = jnp.where(lane == k, arg, iacc)
        s_cur = jnp.where(idx0 == arg, -jnp.inf, s_cur)
        return s_cur, vacc, iacc

    _, vacc, iacc = jax.lax.fori_loop(
        0, NN, step,
        (s, jnp.zeros((1, 128), jnp.float32), jnp.zeros((1, 128), jnp.int32)),
    )
    vals_ref[...] = vacc[:, :NN]
    idx_ref[...] = iacc[:, :NN]


def _topk(scores):
    vals, idx = pl.pallas_call(
        _topk_body,
        in_specs=[pl.BlockSpec((ROWS, COLS), lambda: (0, 0))],
        out_specs=[
            pl.BlockSpec((1, NN), lambda: (0, 0)),
            pl.BlockSpec((1, NN), lambda: (0, 0)),
        ],
        out_shape=[
            jax.ShapeDtypeStruct((1, NN), jnp.float32),
            jax.ShapeDtypeStruct((1, NN), jnp.int32),
        ],
    )(scores)
    return vals[0], idx[0]


def kernel(x, adj, W1, b1, W2, b2, W3, b3, W4, b4, W5, b5,
           Wl1, bl1, Wl2, bl2, Wl3, bl3):
    h0 = x[:, :D]
    x_hi = x[:, D:]
    r, adj_bf = _gcn_layer1(h0, adj, W1, b1.reshape(1, D))
    r = _gcn_layer(r, adj_bf, W2, b2.reshape(1, D))
    r = _gcn_layer(r, adj_bf, W3, b3.reshape(1, D))
    r = _gcn_layer(r, adj_bf, W4, b4.reshape(1, D))
    r = _gcn_layer(r, adj_bf, W5, b5.reshape(1, D))
    Wl3p = jnp.concatenate([Wl3, jnp.zeros((2 * D, 2 * D - 1), jnp.float32)],
                           axis=1)
    scores = _mlp_head(r, x_hi, Wl1, bl1.reshape(1, 256), Wl2,
                       bl2.reshape(1, 2 * D), Wl3p, bl3.reshape(1, 1))
    return _topk(scores.reshape(ROWS, COLS))


# speedup vs baseline: 1.1615x; 1.0512x over previous
import functools

import jax
import jax.numpy as jnp
from jax.experimental import pallas as pl
from jax.experimental.pallas import tpu as pltpu

N = 10000
D = 64
BR1 = 400
BR = 1000
EPS = 1e-5
NN = 70
BF = jnp.bfloat16


def _compute_y(r_ref, w_ref, m_ref, v_ref, y_ref, normalize):
    r = r_ref[...]
    if normalize:
        h = (r - m_ref[...]) / jnp.sqrt(v_ref[...] + EPS)
    else:
        h = r
    y_ref[...] = jnp.dot(h.astype(BF), w_ref[...].astype(BF),
                         preferred_element_type=jnp.float32)


def _layer1_body(r_ref, adj_ref, w_ref, b_ref, out_ref, adjbf_ref, y_ref):
    @pl.when(pl.program_id(0) == 0)
    def _():
        _compute_y(r_ref, w_ref, None, None, y_ref, normalize=False)

    a16 = adj_ref[...].astype(BF)
    adjbf_ref[...] = a16
    out_ref[...] = jnp.maximum(
        jnp.dot(a16, y_ref[...].astype(BF),
                preferred_element_type=jnp.float32) + b_ref[...],
        0.0,
    )


def _layer_body(r_ref, m_ref, v_ref, adj_ref, w_ref, b_ref, out_ref, y_ref):
    @pl.when(pl.program_id(0) == 0)
    def _():
        _compute_y(r_ref, w_ref, m_ref, v_ref, y_ref, normalize=True)

    out_ref[...] = jnp.maximum(
        jnp.dot(adj_ref[...], y_ref[...].astype(BF),
                preferred_element_type=jnp.float32) + b_ref[...],
        0.0,
    )


def _gcn_layer1(h0, adj, W, b):
    return pl.pallas_call(
        _layer1_body,
        grid=(N // BR1,),
        in_specs=[
            pl.BlockSpec((N, D), lambda i: (0, 0)),
            pl.BlockSpec((BR1, N), lambda i: (i, 0)),
            pl.BlockSpec((D, D), lambda i: (0, 0)),
            pl.BlockSpec((1, D), lambda i: (0, 0)),
        ],
        out_specs=[
            pl.BlockSpec((BR1, D), lambda i: (i, 0)),
            pl.BlockSpec((BR1, N), lambda i: (i, 0)),
        ],
        out_shape=[
            jax.ShapeDtypeStruct((N, D), jnp.float32),
            jax.ShapeDtypeStruct((N, N), BF),
        ],
        scratch_shapes=[pltpu.VMEM((N, D), jnp.float32)],
    )(h0, adj, W, b)


def _gcn_layer(r_prev, adj_bf, W, b):
    m = jnp.mean(r_prev, axis=0, keepdims=True)
    v = jnp.var(r_prev, axis=0, keepdims=True)
    return pl.pallas_call(
        _layer_body,
        grid=(N // BR,),
        in_specs=[
            pl.BlockSpec((N, D), lambda i: (0, 0)),
            pl.BlockSpec((1, D), lambda i: (0, 0)),
            pl.BlockSpec((1, D), lambda i: (0, 0)),
            pl.BlockSpec((BR, N), lambda i: (i, 0)),
            pl.BlockSpec((D, D), lambda i: (0, 0)),
            pl.BlockSpec((1, D), lambda i: (0, 0)),
        ],
        out_specs=pl.BlockSpec((BR, D), lambda i: (i, 0)),
        out_shape=jax.ShapeDtypeStruct((N, D), jnp.float32),
        scratch_shapes=[pltpu.VMEM((N, D), jnp.float32)],
    )(r_prev, m, v, adj_bf, W, b)


BM = 2000
NMBLK = N // BM


def _mlp_body(r_ref, x_ref, w1_ref, b1_ref, w2_ref, b2_ref, w3_ref, b3_ref,
              out_ref):
    hcat = jnp.concatenate([r_ref[...], x_ref[...]], axis=1)
    h1 = jnp.maximum(
        jnp.dot(hcat.astype(BF), w1_ref[...].astype(BF),
                preferred_element_type=jnp.float32) + b1_ref[...],
        0.0,
    )
    h2 = jnp.maximum(
        jnp.dot(h1.astype(BF), w2_ref[...].astype(BF),
                preferred_element_type=jnp.float32) + b2_ref[...],
        0.0,
    )
    s = jnp.dot(h2.astype(BF), w3_ref[...].astype(BF),
                preferred_element_type=jnp.float32)
    out_ref[...] = s[:, :1] + b3_ref[...]


def _mlp_head(r5, x_hi, Wl1, bl1, Wl2, bl2, Wl3p, bl3):
    return pl.pallas_call(
        _mlp_body,
        grid=(NMBLK,),
        in_specs=[
            pl.BlockSpec((BM, D), lambda i: (i, 0)),
            pl.BlockSpec((BM, D), lambda i: (i, 0)),
            pl.BlockSpec((2 * D, 256), lambda i: (0, 0)),
            pl.BlockSpec((1, 256), lambda i: (0, 0)),
            pl.BlockSpec((256, 2 * D), lambda i: (0, 0)),
            pl.BlockSpec((1, 2 * D), lambda i: (0, 0)),
            pl.BlockSpec((2 * D, 2 * D), lambda i: (0, 0)),
            pl.BlockSpec((1, 1), lambda i: (0, 0)),
        ],
        out_specs=pl.BlockSpec((BM, 1), lambda i: (i, 0)),
        out_shape=jax.ShapeDtypeStruct((N, 1), jnp.float32),
    )(r5, x_hi, Wl1, bl1, Wl2, bl2, Wl3p, bl3)


ROWS = 8
COLS = N // ROWS


def _topk_body(s_ref, vals_ref, idx_ref):
    s = s_ref[...]
    iota_r = jax.lax.broadcasted_iota(jnp.int32, (ROWS, COLS), 0)
    iota_c = jax.lax.broadcasted_iota(jnp.int32, (ROWS, COLS), 1)
    idx0 = iota_r * COLS + iota_c
    lane = jax.lax.broadcasted_iota(jnp.int32, (1, 128), 1)

    def step(k, carry):
        s_cur, vacc, iacc = carry
        mx = jnp.max(s_cur)
        cand = jnp.where(s_cur == mx, idx0, jnp.int32(2**31 - 1))
        arg = jnp.min(cand)
        vacc = jnp.where(lane == k, mx, vacc)
        iacc = jnp.where(lane == k, arg, iacc)
        s_cur = jnp.where(idx0 == arg, -jnp.inf, s_cur)
        return s_cur, vacc, iacc

    _, vacc, iacc = jax.lax.fori_loop(
        0, NN, step,
        (s, jnp.zeros((1, 128), jnp.float32), jnp.zeros((1, 128), jnp.int32)),
    )
    vals_ref[...] = vacc[:, :NN]
    idx_ref[...] = iacc[:, :NN]


def _topk(scores):
    vals, idx = pl.pallas_call(
        _topk_body,
        in_specs=[pl.BlockSpec((ROWS, COLS), lambda: (0, 0))],
        out_specs=[
            pl.BlockSpec((1, NN), lambda: (0, 0)),
            pl.BlockSpec((1, NN), lambda: (0, 0)),
        ],
        out_shape=[
            jax.ShapeDtypeStruct((1, NN), jnp.float32),
            jax.ShapeDtypeStruct((1, NN), jnp.int32),
        ],
    )(scores)
    return vals[0], idx[0]


def kernel(x, adj, W1, b1, W2, b2, W3, b3, W4, b4, W5, b5,
           Wl1, bl1, Wl2, bl2, Wl3, bl3):
    h0 = x[:, :D]
    x_hi = x[:, D:]
    r, adj_bf = _gcn_layer1(h0, adj, W1, b1.reshape(1, D))
    r = _gcn_layer(r, adj_bf, W2, b2.reshape(1, D))
    r = _gcn_layer(r, adj_bf, W3, b3.reshape(1, D))
    r = _gcn_layer(r, adj_bf, W4, b4.reshape(1, D))
    r = _gcn_layer(r, adj_bf, W5, b5.reshape(1, D))
    Wl3p = jnp.concatenate([Wl3, jnp.zeros((2 * D, 2 * D - 1), jnp.float32)],
                           axis=1)
    scores = _mlp_head(r, x_hi, Wl1, bl1.reshape(1, 256), Wl2,
                       bl2.reshape(1, 2 * D), Wl3p, bl3.reshape(1, 1))
    return _topk(scores.reshape(ROWS, COLS))
